# baseline (speedup 1.0000x reference)
import functools

import jax
import jax.numpy as jnp
from jax import lax
from jax.experimental import pallas as pl
from jax.experimental.pallas import tpu as pltpu
from jax.experimental.pallas import tpu_sc as plsc

VOCAB = 1000000
MAX_LEN = 200
EMBED_DIM = 64
BATCH = 1024
SEQ_LEN = 200

N = BATCH * SEQ_LEN
_INFO = plsc.get_sparse_core_info()
NC = _INFO.num_cores
NS = _INFO.num_subcores
NW = NC * NS
PER_W = N // NW
CHUNK = 128
NCHUNK = PER_W // CHUNK
LANES = 16
VPR = EMBED_DIM // LANES

_mesh = plsc.VectorSubcoreMesh(core_axis_name="c", subcore_axis_name="s")


@functools.partial(
    pl.kernel,
    out_type=jax.ShapeDtypeStruct((N, EMBED_DIM), jnp.float32),
    mesh=_mesh,
    compiler_params=pltpu.CompilerParams(use_tc_tiling_on_sc=False),
    scratch_types=[
        pltpu.VMEM((PER_W,), jnp.int32),
        pltpu.VMEM((2 * MAX_LEN, EMBED_DIM), jnp.float32),
        pltpu.VMEM((CHUNK, EMBED_DIM), jnp.float32),
        pltpu.SemaphoreType.DMA,
    ],
)
def _embed_sc(idx_hbm, tok_hbm, pos_hbm, out_hbm, idx_v, pos_v, data_v, sem):
    wid = lax.axis_index("s") * NC + lax.axis_index("c")
    base = wid * PER_W

    pltpu.sync_copy(idx_hbm.at[pl.ds(base, PER_W)], idx_v)
    pltpu.sync_copy(pos_hbm, pos_v.at[pl.ds(0, MAX_LEN)])
    pltpu.sync_copy(pos_hbm, pos_v.at[pl.ds(MAX_LEN, MAX_LEN)])

    def chunk_body(c, carry):
        pltpu.async_copy(
            tok_hbm.at[idx_v.at[pl.ds(c * CHUNK, CHUNK)]], data_v, sem
        ).wait()
        pos_base = lax.rem(c * CHUNK, MAX_LEN)

        def row_body(r, carry2):
            pr = pos_base + r
            for j in range(VPR):
                sl = pl.ds(j * LANES, LANES)
                data_v[r, sl] = data_v[r, sl] + pos_v[pr, sl]
            return carry2

        lax.fori_loop(0, CHUNK, row_body, 0, unroll=2)
        pltpu.sync_copy(data_v, out_hbm.at[pl.ds(base + c * CHUNK, CHUNK)])
        return carry

    lax.fori_loop(0, NCHUNK, chunk_body, 0)


def kernel(inputs, token_table, position_table):
    flat = inputs.reshape(N)
    out = _embed_sc(flat, token_table, position_table)
    return out.reshape(BATCH, SEQ_LEN, EMBED_DIM)


# speedup vs baseline: 1.0796x; 1.0796x over previous
import functools

import jax
import jax.numpy as jnp
from jax import lax
from jax.experimental import pallas as pl
from jax.experimental.pallas import tpu as pltpu
from jax.experimental.pallas import tpu_sc as plsc

VOCAB = 1000000
MAX_LEN = 200
EMBED_DIM = 64
BATCH = 1024
SEQ_LEN = 200

N = BATCH * SEQ_LEN
_INFO = plsc.get_sparse_core_info()
NC = _INFO.num_cores
NS = _INFO.num_subcores
NW = NC * NS
PER_W = N // NW
CHUNK = 128
NCHUNK = PER_W // CHUNK
LANES = 16
VPR = EMBED_DIM // LANES
NBUF = 5
SKEW = 4

_mesh = plsc.VectorSubcoreMesh(core_axis_name="c", subcore_axis_name="s")


@functools.partial(
    pl.kernel,
    out_type=jax.ShapeDtypeStruct((N, EMBED_DIM), jnp.float32),
    mesh=_mesh,
    compiler_params=pltpu.CompilerParams(use_tc_tiling_on_sc=False),
    scratch_types=[
        pltpu.VMEM((PER_W,), jnp.int32),
        pltpu.VMEM((2 * MAX_LEN, EMBED_DIM), jnp.float32),
        [pltpu.VMEM((CHUNK, EMBED_DIM), jnp.float32) for _ in range(NBUF)],
        [pltpu.SemaphoreType.DMA for _ in range(NBUF)],
        [pltpu.SemaphoreType.DMA for _ in range(NBUF)],
    ],
)
def _embed_sc(idx_hbm, tok_hbm, pos_hbm, out_hbm, idx_v, pos_v, data, gsem, ssem):
    wid = lax.axis_index("s") * NC + lax.axis_index("c")
    base = wid * PER_W

    pltpu.sync_copy(idx_hbm.at[pl.ds(base, PER_W)], idx_v)
    pltpu.sync_copy(pos_hbm, pos_v.at[pl.ds(0, MAX_LEN)])
    pltpu.sync_copy(pos_hbm, pos_v.at[pl.ds(MAX_LEN, MAX_LEN)])

    def gather_start(c, b):
        pltpu.make_async_copy(
            tok_hbm.at[idx_v.at[pl.ds(c * CHUNK, CHUNK)]], data[b], gsem[b]
        ).start()

    def gather_wait(b):
        pltpu.make_async_copy(
            tok_hbm.at[idx_v.at[pl.ds(0, CHUNK)]], data[b], gsem[b]
        ).wait()

    def scatter_start(c, b):
        pltpu.make_async_copy(
            data[b], out_hbm.at[pl.ds(base + c * CHUNK, CHUNK)], ssem[b]
        ).start()

    def scatter_wait(b):
        pltpu.make_async_copy(
            data[b], out_hbm.at[pl.ds(base, CHUNK)], ssem[b]
        ).wait()

    for b in range(SKEW):
        gather_start(b, b)

    def outer(g, carry):
        for b in range(NBUF):
            c = NBUF * g + b
            gather_wait(b)
            pos_base = lax.rem(c * CHUNK, MAX_LEN)

            def row_body(r, carry2):
                pr = pos_base + r
                for j in range(VPR):
                    sl = pl.ds(j * LANES, LANES)
                    data[b][r, sl] = data[b][r, sl] + pos_v[pr, sl]
                return carry2

            lax.fori_loop(0, CHUNK, row_body, 0, unroll=4)
            scatter_start(c, b)

            f = c + SKEW
            bf = (b + SKEW) % NBUF

            @pl.when(jnp.logical_and(c >= 1, f < NCHUNK))
            def _():
                scatter_wait(bf)

            @pl.when(f < NCHUNK)
            def _():
                gather_start(f, bf)

        return carry

    lax.fori_loop(0, NCHUNK // NBUF, outer, 0)

    for b in range(NBUF):
        scatter_wait(b)


def kernel(inputs, token_table, position_table):
    flat = inputs.reshape(N)
    out = _embed_sc(flat, token_table, position_table)
    return out.reshape(BATCH, SEQ_LEN, EMBED_DIM)
